# speedup vs baseline: 1.4513x; 1.4513x over previous
import jax
import jax.numpy as jnp
from jax.experimental import pallas as pl
from jax.experimental.pallas import tpu as pltpu

_B = 128
_N = 32768
_TOP_K = 8
_N_STEPS = 5
_ROWS = 8


def _dg_body(phase_ref, amp_ref, scale_ref, out_ref):
    scale = jnp.maximum(scale_ref[0], 0.01)
    rate = (amp_ref[...] * scale) * (0.5 * (1.0 + jnp.cos(2.0 * jnp.pi * phase_ref[...])))
    alpha = 2.0 / (_N_STEPS + 1.0)
    ema = jnp.zeros_like(rate)
    for _ in range(_N_STEPS):
        ema = alpha * rate + (1.0 - alpha) * ema

    work = ema
    col = jax.lax.broadcasted_iota(jnp.int32, ema.shape, 1)
    mask = jnp.zeros_like(ema, dtype=jnp.bool_)
    for _ in range(_TOP_K):
        m = jnp.max(work, axis=1, keepdims=True)
        eq = work == m
        idx = jnp.min(jnp.where(eq, col, _N), axis=1, keepdims=True)
        hit = col == idx
        mask = jnp.logical_or(mask, hit)
        work = jnp.where(hit, -jnp.inf, work)

    out_ref[...] = jnp.where(mask, ema, 0.0)


@jax.jit
def kernel(phase, amplitude, ffi_scale, fbi_temperature):
    del fbi_temperature
    grid = (_B // _ROWS,)
    return pl.pallas_call(
        _dg_body,
        grid=grid,
        in_specs=[
            pl.BlockSpec((_ROWS, _N), lambda i: (i, 0)),
            pl.BlockSpec((_ROWS, _N), lambda i: (i, 0)),
            pl.BlockSpec(memory_space=pltpu.SMEM),
        ],
        out_specs=pl.BlockSpec((_ROWS, _N), lambda i: (i, 0)),
        out_shape=jax.ShapeDtypeStruct((_B, _N), jnp.float32),
    )(phase, amplitude, jnp.reshape(ffi_scale, (1,)))


# speedup vs baseline: 2.0599x; 1.4193x over previous
import jax
import jax.numpy as jnp
from jax.experimental import pallas as pl
from jax.experimental.pallas import tpu as pltpu

_B = 128
_N = 32768
_TOP_K = 8
_N_STEPS = 5
_ROWS = 8


def _exact_mask(ema):
    work = ema
    col = jax.lax.broadcasted_iota(jnp.int32, ema.shape, 1)
    mask = jnp.zeros_like(ema, dtype=jnp.bool_)
    for _ in range(_TOP_K):
        m = jnp.max(work, axis=1, keepdims=True)
        key = jnp.where(work == m, col, _N)
        idx = jnp.min(key, axis=1, keepdims=True)
        hit = key == idx
        mask = jnp.logical_or(mask, hit)
        work = jnp.where(hit, -1.0, work)
    return mask


def _dg_body(phase_ref, amp_ref, scale_ref, out_ref):
    scale = jnp.maximum(scale_ref[0], 0.01)
    rate = (amp_ref[...] * scale) * (0.5 * (1.0 + jnp.cos(2.0 * jnp.pi * phase_ref[...])))
    alpha = 2.0 / (_N_STEPS + 1.0)
    ema = jnp.zeros_like(rate)
    for _ in range(_N_STEPS):
        ema = alpha * rate + (1.0 - alpha) * ema

    work = ema
    for _ in range(_TOP_K):
        m = jnp.max(work, axis=1, keepdims=True)
        work = jnp.where(work == m, -1.0, work)
    sel = work != ema
    cnt = jnp.sum(sel.astype(jnp.float32), axis=1)
    good = jnp.all(cnt == float(_TOP_K))

    @pl.when(good)
    def _():
        out_ref[...] = jnp.where(sel, ema, 0.0)

    @pl.when(jnp.logical_not(good))
    def _():
        out_ref[...] = jnp.where(_exact_mask(ema), ema, 0.0)


@jax.jit
def kernel(phase, amplitude, ffi_scale, fbi_temperature):
    del fbi_temperature
    grid = (_B // _ROWS,)
    return pl.pallas_call(
        _dg_body,
        grid=grid,
        in_specs=[
            pl.BlockSpec((_ROWS, _N), lambda i: (i, 0)),
            pl.BlockSpec((_ROWS, _N), lambda i: (i, 0)),
            pl.BlockSpec(memory_space=pltpu.SMEM),
        ],
        out_specs=pl.BlockSpec((_ROWS, _N), lambda i: (i, 0)),
        out_shape=jax.ShapeDtypeStruct((_B, _N), jnp.float32),
    )(phase, amplitude, jnp.reshape(ffi_scale, (1,)))


# speedup vs baseline: 2.7074x; 1.3143x over previous
import math

import jax
import jax.numpy as jnp
from jax.experimental import pallas as pl
from jax.experimental.pallas import tpu as pltpu

_B = 128
_N = 32768
_TOP_K = 8
_N_STEPS = 5
_ROWS = 8

_COS_COEFFS = tuple(
    (-1.0) ** k * (2.0 * math.pi) ** (2 * k) / math.factorial(2 * k)
    for k in range(8)
)


def _cos2pi(phi):
    r = phi - jnp.round(phi)
    a = jnp.abs(r)
    flip = a > 0.25
    b = jnp.where(flip, 0.5 - a, a)
    t = b * b
    acc = jnp.full_like(t, _COS_COEFFS[7])
    for k in range(6, -1, -1):
        acc = acc * t + _COS_COEFFS[k]
    return jnp.where(flip, -acc, acc)


def _exact_mask(ema):
    work = ema
    col = jax.lax.broadcasted_iota(jnp.int32, ema.shape, 1)
    mask = jnp.zeros_like(ema, dtype=jnp.bool_)
    for _ in range(_TOP_K):
        m = jnp.max(work, axis=1, keepdims=True)
        key = jnp.where(work == m, col, _N)
        idx = jnp.min(key, axis=1, keepdims=True)
        hit = key == idx
        mask = jnp.logical_or(mask, hit)
        work = jnp.where(hit, -1.0, work)
    return mask


def _dg_body(phase_ref, amp_ref, scale_ref, out_ref):
    scale = jnp.maximum(scale_ref[0], 0.01)
    rate = (amp_ref[...] * scale) * (0.5 * (1.0 + _cos2pi(phase_ref[...])))
    alpha = 2.0 / (_N_STEPS + 1.0)
    ema = jnp.zeros_like(rate)
    for _ in range(_N_STEPS):
        ema = alpha * rate + (1.0 - alpha) * ema

    work = ema
    for _ in range(_TOP_K):
        m = jnp.max(work, axis=1, keepdims=True)
        work = jnp.where(work == m, -1.0, work)
    sel = work != ema
    cnt = jnp.sum(sel.astype(jnp.float32), axis=1)
    good = jnp.all(cnt == float(_TOP_K))

    @pl.when(good)
    def _():
        out_ref[...] = jnp.where(sel, ema, 0.0)

    @pl.when(jnp.logical_not(good))
    def _():
        out_ref[...] = jnp.where(_exact_mask(ema), ema, 0.0)


@jax.jit
def kernel(phase, amplitude, ffi_scale, fbi_temperature):
    del fbi_temperature
    grid = (_B // _ROWS,)
    return pl.pallas_call(
        _dg_body,
        grid=grid,
        in_specs=[
            pl.BlockSpec((_ROWS, _N), lambda i: (i, 0)),
            pl.BlockSpec((_ROWS, _N), lambda i: (i, 0)),
            pl.BlockSpec(memory_space=pltpu.SMEM),
        ],
        out_specs=pl.BlockSpec((_ROWS, _N), lambda i: (i, 0)),
        out_shape=jax.ShapeDtypeStruct((_B, _N), jnp.float32),
    )(phase, amplitude, jnp.reshape(ffi_scale, (1,)))


# speedup vs baseline: 3.5898x; 1.3259x over previous
import math

import jax
import jax.numpy as jnp
from jax.experimental import pallas as pl
from jax.experimental.pallas import tpu as pltpu

_B = 128
_N = 32768
_TOP_K = 8
_N_STEPS = 5
_ROWS = 16

_COS_COEFFS = tuple(
    (-1.0) ** k * (2.0 * math.pi) ** (2 * k) / math.factorial(2 * k)
    for k in range(8)
)


def _cos2pi(phi):
    r = phi - jnp.round(phi)
    a = jnp.abs(r)
    flip = a > 0.25
    b = jnp.where(flip, 0.5 - a, a)
    t = b * b
    acc = jnp.full_like(t, _COS_COEFFS[7])
    for k in range(6, -1, -1):
        acc = acc * t + _COS_COEFFS[k]
    return jnp.where(flip, -acc, acc)


def _exact_mask(ema):
    work = ema
    col = jax.lax.broadcasted_iota(jnp.int32, ema.shape, 1)
    mask = jnp.zeros_like(ema, dtype=jnp.bool_)
    for _ in range(_TOP_K):
        m = jnp.max(work, axis=1, keepdims=True)
        key = jnp.where(work == m, col, _N)
        idx = jnp.min(key, axis=1, keepdims=True)
        hit = key == idx
        mask = jnp.logical_or(mask, hit)
        work = jnp.where(hit, -1.0, work)
    return mask


def _dg_body(phase_ref, amp_ref, scale_ref, out_ref):
    scale = jnp.maximum(scale_ref[0], 0.01)
    rate = (amp_ref[...] * scale) * (0.5 * (1.0 + _cos2pi(phase_ref[...])))
    alpha = 2.0 / (_N_STEPS + 1.0)
    ema = jnp.zeros_like(rate)
    for _ in range(_N_STEPS):
        ema = alpha * rate + (1.0 - alpha) * ema

    work = ema
    for _ in range(_TOP_K):
        m = jnp.max(work, axis=1, keepdims=True)
        work = jnp.where(work == m, -1.0, work)
    sel = work != ema
    cnt = jnp.sum(sel.astype(jnp.float32), axis=1)
    good = jnp.all(cnt == float(_TOP_K))

    @pl.when(good)
    def _():
        out_ref[...] = jnp.where(sel, ema, 0.0)

    @pl.when(jnp.logical_not(good))
    def _():
        out_ref[...] = jnp.where(_exact_mask(ema), ema, 0.0)


@jax.jit
def kernel(phase, amplitude, ffi_scale, fbi_temperature):
    del fbi_temperature
    grid = (_B // _ROWS,)
    return pl.pallas_call(
        _dg_body,
        grid=grid,
        in_specs=[
            pl.BlockSpec((_ROWS, _N), lambda i: (i, 0)),
            pl.BlockSpec((_ROWS, _N), lambda i: (i, 0)),
            pl.BlockSpec(memory_space=pltpu.SMEM),
        ],
        out_specs=pl.BlockSpec((_ROWS, _N), lambda i: (i, 0)),
        out_shape=jax.ShapeDtypeStruct((_B, _N), jnp.float32),
    )(phase, amplitude, jnp.reshape(ffi_scale, (1,)))


# speedup vs baseline: 3.7380x; 1.0413x over previous
import math

import jax
import jax.numpy as jnp
from jax.experimental import pallas as pl
from jax.experimental.pallas import tpu as pltpu

_B = 128
_N = 32768
_TOP_K = 8
_N_STEPS = 5
_ROWS = 16

_COS_COEFFS = tuple(
    (-1.0) ** k * (2.0 * math.pi) ** (2 * k) / math.factorial(2 * k)
    for k in range(8)
)


def _cos2pi(phi):
    r = phi - jnp.round(phi)
    a = jnp.abs(r)
    flip = a > 0.25
    b = jnp.where(flip, 0.5 - a, a)
    t = b * b
    acc = jnp.full_like(t, _COS_COEFFS[7])
    for k in range(6, -1, -1):
        acc = acc * t + _COS_COEFFS[k]
    return jnp.where(flip, -acc, acc)


def _exact_mask(ema):
    work = ema
    col = jax.lax.broadcasted_iota(jnp.int32, ema.shape, 1)
    mask = jnp.zeros_like(ema, dtype=jnp.bool_)
    for _ in range(_TOP_K):
        m = jnp.max(work, axis=1, keepdims=True)
        key = jnp.where(work == m, col, _N)
        idx = jnp.min(key, axis=1, keepdims=True)
        hit = key == idx
        mask = jnp.logical_or(mask, hit)
        work = jnp.where(hit, -1.0, work)
    return mask


def _dg_body(phase_ref, amp_ref, scale_ref, out_ref):
    scale = jnp.maximum(scale_ref[0], 0.01)
    rate = (amp_ref[...] * scale) * (0.5 * (1.0 + _cos2pi(phase_ref[...])))
    alpha = 2.0 / (_N_STEPS + 1.0)
    ema = jnp.zeros_like(rate)
    for _ in range(_N_STEPS):
        ema = alpha * rate + (1.0 - alpha) * ema

    tops = [jnp.full((ema.shape[0], 128), -1.0, dtype=jnp.float32) for _ in range(_TOP_K)]
    for v in range(_N // 128):
        x = ema[:, v * 128:(v + 1) * 128]
        for k in range(_TOP_K):
            t = tops[k]
            tops[k] = jnp.maximum(t, x)
            x = jnp.minimum(t, x)
    cand = jnp.concatenate(tops, axis=1)

    for _ in range(_TOP_K):
        m = jnp.max(cand, axis=1, keepdims=True)
        cand = jnp.where(cand == m, -1.0, cand)
    sel = ema >= m
    cnt = jnp.sum(sel.astype(jnp.float32), axis=1)
    good = jnp.all(cnt == float(_TOP_K))

    @pl.when(good)
    def _():
        out_ref[...] = jnp.where(sel, ema, 0.0)

    @pl.when(jnp.logical_not(good))
    def _():
        out_ref[...] = jnp.where(_exact_mask(ema), ema, 0.0)


@jax.jit
def kernel(phase, amplitude, ffi_scale, fbi_temperature):
    del fbi_temperature
    grid = (_B // _ROWS,)
    return pl.pallas_call(
        _dg_body,
        grid=grid,
        in_specs=[
            pl.BlockSpec((_ROWS, _N), lambda i: (i, 0)),
            pl.BlockSpec((_ROWS, _N), lambda i: (i, 0)),
            pl.BlockSpec(memory_space=pltpu.SMEM),
        ],
        out_specs=pl.BlockSpec((_ROWS, _N), lambda i: (i, 0)),
        out_shape=jax.ShapeDtypeStruct((_B, _N), jnp.float32),
    )(phase, amplitude, jnp.reshape(ffi_scale, (1,)))
